# baseline (speedup 1.0000x reference)
import jax
import jax.numpy as jnp
from jax.experimental import pallas as pl


def kernel(input, nodes, edges, count2label, conv_w, gru_wi, gru_wh, gru_bi, gru_bh, Wi, bi, Wj, bj, Wm, bm, Wv, bv, Wa, ba):
    inp_x = input
    Bn = inp_x.shape[0]
    Cn = count2label.shape[0]
    Nn = nodes.shape[0]
    cov = inp_x[:, :Cn]
    winner = jnp.zeros((Nn,), jnp.int32).at[count2label].max(
        jnp.arange(1, Cn + 1, dtype=jnp.int32))
    gathered = jnp.take(cov, jnp.maximum(winner - 1, 0), axis=1)
    covAll = jnp.where(winner[None, :] > 0, gathered, 0.0)
    nodesInput = jnp.concatenate(
        [jnp.broadcast_to(nodes[None], (Bn, Nn, nodes.shape[1])), covAll[:, :, None]],
        axis=2).reshape(Bn * Nn, -1)
    offsets = (jnp.arange(Bn) * Nn).astype(edges.dtype)
    edgesInput = (edges[:, None, :] + offsets[None, :, None]).reshape(2, -1)
    src, dst = edgesInput[0], edgesInput[1]
    x = nodesInput
    for i in range(conv_w.shape[0]):
        m = x @ conv_w[i]
        m_agg = jax.ops.segment_sum(m[src], dst, num_segments=Bn * Nn)
        gi = m_agg @ gru_wi + gru_bi
        gh = x @ gru_wh + gru_bh
        i_r, i_z, i_n = jnp.split(gi, 3, axis=1)
        h_r, h_z, h_n = jnp.split(gh, 3, axis=1)
        r = jax.nn.sigmoid(i_r + h_r)
        z = jax.nn.sigmoid(i_z + h_z)
        n = jnp.tanh(i_n + r * h_n)
        x = (1 - z) * n + z * x
    h = jax.nn.relu(x @ Wm + bm)
    xc = jnp.concatenate([h, nodesInput], axis=1)
    att = jax.nn.sigmoid(xc @ Wi + bi) * jax.nn.relu(xc @ Wj + bj)
    batch_ids = jnp.repeat(jnp.arange(Bn), Nn)
    pooled = jax.nn.relu(jax.ops.segment_sum(att, batch_ids, num_segments=Bn))
    value = pooled @ Wv + bv
    advantage = pooled @ Wa + ba
    return value + advantage - jnp.mean(advantage)


# speedup vs baseline: 9.2768x; 9.2768x over previous
import functools

import jax
import jax.numpy as jnp
from jax import lax
from jax.experimental import pallas as pl
from jax.experimental.pallas import tpu as pltpu
from jax.experimental.pallas import tpu_sc as plsc

NT = 16
NCC = 2
CH = 128
SUPER = 16
ZR = 64


def _seg_sum_sc(m, src3, dst3, np_rows, n_nodes, n_batch):
    BN, H = m.shape
    nchunk = src3.shape[1]
    rpt = np_rows // NT
    bpc = n_batch // NCC

    mesh = plsc.VectorSubcoreMesh(core_axis_name="c", subcore_axis_name="s",
                                  num_cores=NCC, num_subcores=NT)

    @functools.partial(
        pl.kernel,
        out_type=jax.ShapeDtypeStruct((n_batch * np_rows, H), jnp.float32),
        mesh=mesh,
        scratch_types=[
            pltpu.VMEM_SHARED((np_rows, H), jnp.float32),
            pltpu.VMEM((SUPER, CH), jnp.int32),
            pltpu.VMEM((SUPER, CH), jnp.int32),
            pltpu.VMEM((CH, H), jnp.float32),
            pltpu.VMEM((ZR, H), jnp.float32),
        ],
    )
    def k(m_hbm, src_hbm, dst_hbm, out_hbm, acc, src_v, dst_v, rows_v, zbuf):
        cid = lax.axis_index("c")
        sid = lax.axis_index("s")
        r0 = pl.multiple_of(sid * rpt, 8)
        nsuper = nchunk // SUPER

        def zrow(i, c):
            def zcol(kk, c2):
                zbuf[i, pl.ds(kk * 16, 16)] = jnp.zeros((16,), jnp.float32)
                return c2
            return lax.fori_loop(0, H // 16, zcol, c)
        lax.fori_loop(0, ZR, zrow, 0)

        for kb in range(bpc):
            b = cid * bpc + kb
            bias = b * n_nodes
            off = 0
            while off < rpt:
                sz = min(ZR, rpt - off)
                pltpu.sync_copy(zbuf.at[pl.ds(0, sz)],
                                acc.at[pl.ds(pl.multiple_of(r0 + off, 8), sz)])
                off += sz
            plsc.subcore_barrier()

            def super_body(s, carry):
                s0 = pl.multiple_of(s * SUPER, 8)
                pltpu.sync_copy(src_hbm.at[sid, pl.ds(s0, SUPER)], src_v)
                pltpu.sync_copy(dst_hbm.at[sid, pl.ds(s0, SUPER)], dst_v)

                def bias_j(j, c2):
                    def bias_k(kk, c3):
                        sl = pl.ds(kk * 16, 16)
                        src_v[j, sl] = src_v[j, sl] + bias
                        return c3
                    return lax.fori_loop(0, CH // 16, bias_k, c2)
                lax.fori_loop(0, SUPER, bias_j, carry)

                def chunk_body(j, c2):
                    pltpu.sync_copy(m_hbm.at[src_v.at[j]], rows_v)
                    pltpu.sync_copy(rows_v, acc.at[dst_v.at[j]], add=True)
                    return c2
                return lax.fori_loop(0, SUPER, chunk_body, carry)
            lax.fori_loop(0, nsuper, super_body, 0)

            plsc.subcore_barrier()
            off = 0
            while off < rpt:
                sz = min(CH, rpt - off)
                pltpu.sync_copy(acc.at[pl.ds(pl.multiple_of(r0 + off, 8), sz)],
                                rows_v.at[pl.ds(0, sz)])
                pltpu.sync_copy(
                    rows_v.at[pl.ds(0, sz)],
                    out_hbm.at[pl.ds(
                        pl.multiple_of(b * np_rows + r0 + off, 8), sz)])
                off += sz

    return k(m, src3, dst3)


_F32 = jnp.float32


def _tc_prologue(nodes_pad, cov_flat, cw0, n_nodes, n_batch, bm_rows):
    n_rows, H = nodes_pad.shape
    nb = n_nodes // bm_rows
    grid = (n_batch * nb,)

    def body(nodes_ref, cov_ref, cw_ref, x0_ref, m0_ref):
        lane = lax.broadcasted_iota(jnp.int32, (bm_rows, H), 1)
        xb = jnp.where(lane == H - 1, cov_ref[...], nodes_ref[...])
        x0_ref[...] = xb
        m0_ref[...] = jnp.dot(xb, cw_ref[...], preferred_element_type=_F32)

    return pl.pallas_call(
        body,
        grid=grid,
        in_specs=[
            pl.BlockSpec((bm_rows, H), lambda i: (i % nb, 0)),
            pl.BlockSpec((bm_rows, 1), lambda i: (i, 0)),
            pl.BlockSpec((H, H), lambda i: (0, 0)),
        ],
        out_specs=[
            pl.BlockSpec((bm_rows, H), lambda i: (i, 0)),
            pl.BlockSpec((bm_rows, H), lambda i: (i, 0)),
        ],
        out_shape=[
            jax.ShapeDtypeStruct((n_batch * n_nodes, H), _F32),
            jax.ShapeDtypeStruct((n_batch * n_nodes, H), _F32),
        ],
    )(nodes_pad, cov_flat, cw0)


def _gru(ma, xb, wi, wh, bi, bh, H):
    gi = jnp.dot(ma, wi, preferred_element_type=_F32) + bi
    gh = jnp.dot(xb, wh, preferred_element_type=_F32) + bh
    r = jax.nn.sigmoid(gi[:, :H] + gh[:, :H])
    z = jax.nn.sigmoid(gi[:, H:2 * H] + gh[:, H:2 * H])
    n = jnp.tanh(gi[:, 2 * H:] + r * gh[:, 2 * H:])
    return (1.0 - z) * n + z * xb


def _tc_gru_conv(m_agg, x, wi, wh, bi, bh, cw_next, bm_rows):
    BN, H = x.shape
    grid = (BN // bm_rows,)

    def body(ma_ref, x_ref, wi_ref, wh_ref, bi_ref, bh_ref, cw_ref,
             xo_ref, mo_ref):
        xn = _gru(ma_ref[...], x_ref[...], wi_ref[...], wh_ref[...],
                  bi_ref[...], bh_ref[...], H)
        xo_ref[...] = xn
        mo_ref[...] = jnp.dot(xn, cw_ref[...], preferred_element_type=_F32)

    row_spec = pl.BlockSpec((bm_rows, H), lambda i: (i, 0))
    full = lambda shp: pl.BlockSpec(shp, lambda i: (0, 0))
    return pl.pallas_call(
        body,
        grid=grid,
        in_specs=[
            row_spec, row_spec,
            full((H, 3 * H)), full((H, 3 * H)),
            full((1, 3 * H)), full((1, 3 * H)),
            full((H, H)),
        ],
        out_specs=[row_spec, row_spec],
        out_shape=[
            jax.ShapeDtypeStruct((BN, H), _F32),
            jax.ShapeDtypeStruct((BN, H), _F32),
        ],
    )(m_agg, x, wi, wh, bi, bh, cw_next)


def _tc_final(m_agg, x, x0, wi, wh, bi, bh, Wm, bm, Wi, bi2, Wj, bj2,
              Wv, bv, Wa, ba, n_nodes, n_batch, bm_rows):
    BN, H = x.shape
    A = Wa.shape[1]
    nb = n_nodes // bm_rows
    grid = (BN // bm_rows,)
    nsteps = BN // bm_rows

    def body(ma_ref, x_ref, x0_ref, wi_ref, wh_ref, bi_ref, bh_ref,
             wm_ref, bm_ref, wi2_ref, bi2_ref, wj_ref, bj2_ref,
             wv_ref, bv_ref, wa_ref, ba_ref, out_ref, pooled_ref):
        i = pl.program_id(0)

        @pl.when(i == 0)
        def _():
            pooled_ref[...] = jnp.zeros_like(pooled_ref)

        xn = _gru(ma_ref[...], x_ref[...], wi_ref[...], wh_ref[...],
                  bi_ref[...], bh_ref[...], H)
        h = jnp.maximum(
            jnp.dot(xn, wm_ref[...], preferred_element_type=_F32) + bm_ref[...],
            0.0)
        x0b = x0_ref[...]
        a1 = (jnp.dot(h, wi2_ref[:H, :], preferred_element_type=_F32)
              + jnp.dot(x0b, wi2_ref[H:, :], preferred_element_type=_F32)
              + bi2_ref[...])
        a2 = (jnp.dot(h, wj_ref[:H, :], preferred_element_type=_F32)
              + jnp.dot(x0b, wj_ref[H:, :], preferred_element_type=_F32)
              + bj2_ref[...])
        att = jax.nn.sigmoid(a1) * jnp.maximum(a2, 0.0)
        bsum = jnp.sum(att, axis=0, keepdims=True)
        bidx = i // nb
        pooled_ref[pl.ds(bidx, 1), :] = pooled_ref[pl.ds(bidx, 1), :] + bsum

        @pl.when(i == nsteps - 1)
        def _():
            pooled = jnp.maximum(pooled_ref[...], 0.0)
            value = jnp.dot(pooled, wv_ref[...],
                            preferred_element_type=_F32) + bv_ref[...]
            adv = jnp.dot(pooled, wa_ref[...],
                          preferred_element_type=_F32) + ba_ref[...]
            row = lax.broadcasted_iota(jnp.int32, adv.shape, 0)
            adv_mean = jnp.sum(jnp.where(row < n_batch, adv, 0.0)) / (
                n_batch * A)
            out_ref[...] = value + adv - adv_mean

    row_spec = pl.BlockSpec((bm_rows, H), lambda i: (i, 0))
    full = lambda shp: pl.BlockSpec(shp, lambda i: (0, 0))
    out8 = pl.pallas_call(
        body,
        grid=grid,
        in_specs=[
            row_spec, row_spec, row_spec,
            full((H, 3 * H)), full((H, 3 * H)),
            full((1, 3 * H)), full((1, 3 * H)),
            full((H, H)), full((1, H)),
            full((2 * H, 2 * H)), full((1, 2 * H)),
            full((2 * H, 2 * H)), full((1, 2 * H)),
            full((2 * H, 1)), full((1, 1)),
            full((2 * H, A)), full((1, A)),
        ],
        out_specs=full((8, A)),
        out_shape=jax.ShapeDtypeStruct((8, A), _F32),
        scratch_shapes=[pltpu.VMEM((8, 2 * H), _F32)],
    )(m_agg, x, x0, wi, wh, bi, bh, Wm, bm, Wi, bi2, Wj, bj2, Wv, bv, Wa, ba)
    return out8


def kernel(input, nodes, edges, count2label, conv_w, gru_wi, gru_wh,
           gru_bi, gru_bh, Wi, bi, Wj, bj, Wm, bm, Wv, bv, Wa, ba):
    Bn = input.shape[0]
    Cn = count2label.shape[0]
    Nn, F = nodes.shape
    H = F + 1
    E = edges.shape[1]
    Lc = conv_w.shape[0]
    BM = 1000

    cov = input[:, :Cn]
    winner = jnp.zeros((Nn,), jnp.int32).at[count2label].max(
        jnp.arange(1, Cn + 1, dtype=jnp.int32))
    covAll = jnp.where(winner[None, :] > 0,
                       jnp.take(cov, jnp.maximum(winner - 1, 0), axis=1),
                       0.0)
    cov_flat = covAll.reshape(Bn * Nn, 1)
    nodes_pad = jnp.pad(nodes, ((0, 0), (0, 1)))

    ept = E // NT
    nchunk = -(-(-(-ept // CH)) // SUPER) * SUPER
    npad = nchunk * CH - ept
    src = edges[0].reshape(NT, ept)
    dst = edges[1].reshape(NT, ept)
    if npad:
        pad_src = ((jnp.arange(NT, dtype=jnp.int32)[:, None] * 1259
                    + jnp.arange(npad, dtype=jnp.int32)[None, :] * 631) % Nn)
        pad_dst = (Nn + jnp.arange(NT, dtype=jnp.int32)[:, None]
                   + jnp.zeros((1, npad), jnp.int32))
        src = jnp.concatenate([src, pad_src], axis=1)
        dst = jnp.concatenate([dst, pad_dst], axis=1)
    src3 = src.reshape(NT, nchunk, CH)
    dst3 = dst.reshape(NT, nchunk, CH)
    np_rows = NT * (-(-(-(-Nn // NT)) // 8) * 8)

    bi1 = gru_bi.reshape(1, 3 * H)
    bh1 = gru_bh.reshape(1, 3 * H)

    x0, m = _tc_prologue(nodes_pad, cov_flat, conv_w[0], Nn, Bn, BM)
    x = x0
    for i in range(Lc):
        m_agg_p = _seg_sum_sc(m, src3, dst3, np_rows, Nn, Bn)
        m_agg = m_agg_p.reshape(Bn, np_rows, H)[:, :Nn].reshape(Bn * Nn, H)
        if i < Lc - 1:
            x, m = _tc_gru_conv(m_agg, x, gru_wi, gru_wh, bi1, bh1,
                                conv_w[i + 1], BM)
        else:
            out8 = _tc_final(m_agg, x, x0, gru_wi, gru_wh, bi1, bh1,
                             Wm, bm.reshape(1, H),
                             Wi, bi.reshape(1, 2 * H),
                             Wj, bj.reshape(1, 2 * H),
                             Wv, bv.reshape(1, 1),
                             Wa, ba.reshape(1, -1),
                             Nn, Bn, BM)
    return out8[:Bn]


# speedup vs baseline: 11.8092x; 1.2730x over previous
import functools

import jax
import jax.numpy as jnp
from jax import lax
from jax.experimental import pallas as pl
from jax.experimental.pallas import tpu as pltpu
from jax.experimental.pallas import tpu_sc as plsc

NT = 16
NCC = 2
CH = 128
SUPER = 16
ZR = 64


def _seg_sum_sc(m, src3, dst3, np_rows, n_nodes, n_batch):
    BN, H = m.shape
    nchunk = src3.shape[1]
    rpt = np_rows // NT
    bpc = n_batch // NCC

    mesh = plsc.VectorSubcoreMesh(core_axis_name="c", subcore_axis_name="s",
                                  num_cores=NCC, num_subcores=NT)

    @functools.partial(
        pl.kernel,
        out_type=jax.ShapeDtypeStruct((n_batch * np_rows, H), jnp.float32),
        mesh=mesh,
        scratch_types=[
            pltpu.VMEM_SHARED((np_rows, H), jnp.float32),
            pltpu.VMEM((SUPER, CH), jnp.int32),
            pltpu.VMEM((SUPER, CH), jnp.int32),
            pltpu.VMEM((CH, H), jnp.float32),
            pltpu.VMEM((CH, H), jnp.float32),
            pltpu.VMEM((ZR, H), jnp.float32),
            pltpu.SemaphoreType.DMA,
            pltpu.SemaphoreType.DMA,
        ],
    )
    def k(m_hbm, src_hbm, dst_hbm, out_hbm, acc, src_v, dst_v, rows_a, rows_b,
          zbuf, gsem, ssem):
        cid = lax.axis_index("c")
        sid = lax.axis_index("s")
        r0 = pl.multiple_of(sid * rpt, 8)
        nsuper = nchunk // SUPER

        def zrow(i, c):
            def zcol(kk, c2):
                zbuf[i, pl.ds(kk * 16, 16)] = jnp.zeros((16,), jnp.float32)
                return c2
            return lax.fori_loop(0, H // 16, zcol, c)
        lax.fori_loop(0, ZR, zrow, 0)

        for kb in range(bpc):
            b = cid * bpc + kb
            bias = b * n_nodes
            off = 0
            while off < rpt:
                sz = min(ZR, rpt - off)
                pltpu.sync_copy(zbuf.at[pl.ds(0, sz)],
                                acc.at[pl.ds(pl.multiple_of(r0 + off, 8), sz)])
                off += sz
            plsc.subcore_barrier()

            def super_body(s, carry):
                s0 = pl.multiple_of(s * SUPER, 8)
                pltpu.sync_copy(src_hbm.at[sid, pl.ds(s0, SUPER)], src_v)
                pltpu.sync_copy(dst_hbm.at[sid, pl.ds(s0, SUPER)], dst_v)

                def bias_j(j, c2):
                    def bias_k(kk, c3):
                        sl = pl.ds(kk * 16, 16)
                        src_v[j, sl] = src_v[j, sl] + bias
                        return c3
                    return lax.fori_loop(0, CH // 16, bias_k, c2)
                lax.fori_loop(0, SUPER, bias_j, carry)

                rows = (rows_a, rows_b)
                g = pltpu.async_copy(m_hbm.at[src_v.at[0]], rows[0], gsem)
                sprev = None
                for jj in range(SUPER):
                    rb = jj % 2
                    g.wait()
                    if sprev is not None:
                        sprev.wait()
                    if jj + 1 < SUPER:
                        g = pltpu.async_copy(m_hbm.at[src_v.at[jj + 1]],
                                             rows[1 - rb], gsem)
                    sprev = pltpu.async_copy(rows[rb], acc.at[dst_v.at[jj]],
                                             ssem, add=True)
                sprev.wait()
                return carry
            lax.fori_loop(0, nsuper, super_body, 0)

            plsc.subcore_barrier()
            off = 0
            while off < rpt:
                sz = min(CH, rpt - off)
                pltpu.sync_copy(acc.at[pl.ds(pl.multiple_of(r0 + off, 8), sz)],
                                rows_a.at[pl.ds(0, sz)])
                pltpu.sync_copy(
                    rows_a.at[pl.ds(0, sz)],
                    out_hbm.at[pl.ds(
                        pl.multiple_of(b * np_rows + r0 + off, 8), sz)])
                off += sz

    return k(m, src3, dst3)


_F32 = jnp.float32


def _tc_prologue(nodes_pad, cov_flat, cw0, n_nodes, n_batch, bm_rows):
    n_rows, H = nodes_pad.shape
    nb = n_nodes // bm_rows
    grid = (n_batch * nb,)

    def body(nodes_ref, cov_ref, cw_ref, x0_ref, m0_ref):
        lane = lax.broadcasted_iota(jnp.int32, (bm_rows, H), 1)
        xb = jnp.where(lane == H - 1, cov_ref[...], nodes_ref[...])
        x0_ref[...] = xb
        m0_ref[...] = jnp.dot(xb, cw_ref[...], preferred_element_type=_F32)

    return pl.pallas_call(
        body,
        grid=grid,
        in_specs=[
            pl.BlockSpec((bm_rows, H), lambda i: (i % nb, 0)),
            pl.BlockSpec((bm_rows, 1), lambda i: (i, 0)),
            pl.BlockSpec((H, H), lambda i: (0, 0)),
        ],
        out_specs=[
            pl.BlockSpec((bm_rows, H), lambda i: (i, 0)),
            pl.BlockSpec((bm_rows, H), lambda i: (i, 0)),
        ],
        out_shape=[
            jax.ShapeDtypeStruct((n_batch * n_nodes, H), _F32),
            jax.ShapeDtypeStruct((n_batch * n_nodes, H), _F32),
        ],
    )(nodes_pad, cov_flat, cw0)


def _gru(ma, xb, wi, wh, bi, bh, H):
    gi = jnp.dot(ma, wi, preferred_element_type=_F32) + bi
    gh = jnp.dot(xb, wh, preferred_element_type=_F32) + bh
    r = jax.nn.sigmoid(gi[:, :H] + gh[:, :H])
    z = jax.nn.sigmoid(gi[:, H:2 * H] + gh[:, H:2 * H])
    n = jnp.tanh(gi[:, 2 * H:] + r * gh[:, 2 * H:])
    return (1.0 - z) * n + z * xb


def _tc_gru_conv(m_agg, x, wi, wh, bi, bh, cw_next, bm_rows):
    BN, H = x.shape
    grid = (BN // bm_rows,)

    def body(ma_ref, x_ref, wi_ref, wh_ref, bi_ref, bh_ref, cw_ref,
             xo_ref, mo_ref):
        xn = _gru(ma_ref[...], x_ref[...], wi_ref[...], wh_ref[...],
                  bi_ref[...], bh_ref[...], H)
        xo_ref[...] = xn
        mo_ref[...] = jnp.dot(xn, cw_ref[...], preferred_element_type=_F32)

    row_spec = pl.BlockSpec((bm_rows, H), lambda i: (i, 0))
    full = lambda shp: pl.BlockSpec(shp, lambda i: (0, 0))
    return pl.pallas_call(
        body,
        grid=grid,
        in_specs=[
            row_spec, row_spec,
            full((H, 3 * H)), full((H, 3 * H)),
            full((1, 3 * H)), full((1, 3 * H)),
            full((H, H)),
        ],
        out_specs=[row_spec, row_spec],
        out_shape=[
            jax.ShapeDtypeStruct((BN, H), _F32),
            jax.ShapeDtypeStruct((BN, H), _F32),
        ],
    )(m_agg, x, wi, wh, bi, bh, cw_next)


def _tc_final(m_agg, x, x0, wi, wh, bi, bh, Wm, bm, Wi, bi2, Wj, bj2,
              Wv, bv, Wa, ba, n_nodes, n_batch, bm_rows):
    BN, H = x.shape
    A = Wa.shape[1]
    nb = n_nodes // bm_rows
    grid = (BN // bm_rows,)
    nsteps = BN // bm_rows

    def body(ma_ref, x_ref, x0_ref, wi_ref, wh_ref, bi_ref, bh_ref,
             wm_ref, bm_ref, wi2_ref, bi2_ref, wj_ref, bj2_ref,
             wv_ref, bv_ref, wa_ref, ba_ref, out_ref, pooled_ref):
        i = pl.program_id(0)

        @pl.when(i == 0)
        def _():
            pooled_ref[...] = jnp.zeros_like(pooled_ref)

        xn = _gru(ma_ref[...], x_ref[...], wi_ref[...], wh_ref[...],
                  bi_ref[...], bh_ref[...], H)
        h = jnp.maximum(
            jnp.dot(xn, wm_ref[...], preferred_element_type=_F32) + bm_ref[...],
            0.0)
        x0b = x0_ref[...]
        a1 = (jnp.dot(h, wi2_ref[:H, :], preferred_element_type=_F32)
              + jnp.dot(x0b, wi2_ref[H:, :], preferred_element_type=_F32)
              + bi2_ref[...])
        a2 = (jnp.dot(h, wj_ref[:H, :], preferred_element_type=_F32)
              + jnp.dot(x0b, wj_ref[H:, :], preferred_element_type=_F32)
              + bj2_ref[...])
        att = jax.nn.sigmoid(a1) * jnp.maximum(a2, 0.0)
        bsum = jnp.sum(att, axis=0, keepdims=True)
        bidx = i // nb
        pooled_ref[pl.ds(bidx, 1), :] = pooled_ref[pl.ds(bidx, 1), :] + bsum

        @pl.when(i == nsteps - 1)
        def _():
            pooled = jnp.maximum(pooled_ref[...], 0.0)
            value = jnp.dot(pooled, wv_ref[...],
                            preferred_element_type=_F32) + bv_ref[...]
            adv = jnp.dot(pooled, wa_ref[...],
                          preferred_element_type=_F32) + ba_ref[...]
            row = lax.broadcasted_iota(jnp.int32, adv.shape, 0)
            adv_mean = jnp.sum(jnp.where(row < n_batch, adv, 0.0)) / (
                n_batch * A)
            out_ref[...] = value + adv - adv_mean

    row_spec = pl.BlockSpec((bm_rows, H), lambda i: (i, 0))
    full = lambda shp: pl.BlockSpec(shp, lambda i: (0, 0))
    out8 = pl.pallas_call(
        body,
        grid=grid,
        in_specs=[
            row_spec, row_spec, row_spec,
            full((H, 3 * H)), full((H, 3 * H)),
            full((1, 3 * H)), full((1, 3 * H)),
            full((H, H)), full((1, H)),
            full((2 * H, 2 * H)), full((1, 2 * H)),
            full((2 * H, 2 * H)), full((1, 2 * H)),
            full((2 * H, 1)), full((1, 1)),
            full((2 * H, A)), full((1, A)),
        ],
        out_specs=full((8, A)),
        out_shape=jax.ShapeDtypeStruct((8, A), _F32),
        scratch_shapes=[pltpu.VMEM((8, 2 * H), _F32)],
    )(m_agg, x, x0, wi, wh, bi, bh, Wm, bm, Wi, bi2, Wj, bj2, Wv, bv, Wa, ba)
    return out8


def kernel(input, nodes, edges, count2label, conv_w, gru_wi, gru_wh,
           gru_bi, gru_bh, Wi, bi, Wj, bj, Wm, bm, Wv, bv, Wa, ba):
    Bn = input.shape[0]
    Cn = count2label.shape[0]
    Nn, F = nodes.shape
    H = F + 1
    E = edges.shape[1]
    Lc = conv_w.shape[0]
    BM = 1000

    cov = input[:, :Cn]
    winner = jnp.zeros((Nn,), jnp.int32).at[count2label].max(
        jnp.arange(1, Cn + 1, dtype=jnp.int32))
    covAll = jnp.where(winner[None, :] > 0,
                       jnp.take(cov, jnp.maximum(winner - 1, 0), axis=1),
                       0.0)
    cov_flat = covAll.reshape(Bn * Nn, 1)
    nodes_pad = jnp.pad(nodes, ((0, 0), (0, 1)))

    ept = E // NT
    nchunk = -(-(-(-ept // CH)) // SUPER) * SUPER
    npad = nchunk * CH - ept
    src = edges[0].reshape(NT, ept)
    dst = edges[1].reshape(NT, ept)
    if npad:
        pad_src = ((jnp.arange(NT, dtype=jnp.int32)[:, None] * 1259
                    + jnp.arange(npad, dtype=jnp.int32)[None, :] * 631) % Nn)
        pad_dst = (Nn + jnp.arange(NT, dtype=jnp.int32)[:, None]
                   + jnp.zeros((1, npad), jnp.int32))
        src = jnp.concatenate([src, pad_src], axis=1)
        dst = jnp.concatenate([dst, pad_dst], axis=1)
    src3 = src.reshape(NT, nchunk, CH)
    dst3 = dst.reshape(NT, nchunk, CH)
    np_rows = NT * (-(-(-(-Nn // NT)) // 8) * 8)

    bi1 = gru_bi.reshape(1, 3 * H)
    bh1 = gru_bh.reshape(1, 3 * H)

    x0, m = _tc_prologue(nodes_pad, cov_flat, conv_w[0], Nn, Bn, BM)
    x = x0
    for i in range(Lc):
        m_agg_p = _seg_sum_sc(m, src3, dst3, np_rows, Nn, Bn)
        m_agg = m_agg_p.reshape(Bn, np_rows, H)[:, :Nn].reshape(Bn * Nn, H)
        if i < Lc - 1:
            x, m = _tc_gru_conv(m_agg, x, gru_wi, gru_wh, bi1, bh1,
                                conv_w[i + 1], BM)
        else:
            out8 = _tc_final(m_agg, x, x0, gru_wi, gru_wh, bi1, bh1,
                             Wm, bm.reshape(1, H),
                             Wi, bi.reshape(1, 2 * H),
                             Wj, bj.reshape(1, 2 * H),
                             Wv, bv.reshape(1, 1),
                             Wa, ba.reshape(1, -1),
                             Nn, Bn, BM)
    return out8[:Bn]


# speedup vs baseline: 12.1876x; 1.0320x over previous
import functools

import jax
import jax.numpy as jnp
from jax import lax
from jax.experimental import pallas as pl
from jax.experimental.pallas import tpu as pltpu
from jax.experimental.pallas import tpu_sc as plsc

NT = 16
NCC = 2
CH = 128
SUPER = 32
ZR = 64


def _seg_sum_sc(m, src3, dst3, np_rows, n_batch):
    BN, H = m.shape
    nchunk = src3.shape[1]
    rpt = np_rows // NT
    bpc = n_batch // NCC

    mesh = plsc.VectorSubcoreMesh(core_axis_name="c", subcore_axis_name="s",
                                  num_cores=NCC, num_subcores=NT)

    @functools.partial(
        pl.kernel,
        out_type=jax.ShapeDtypeStruct((BN, H), jnp.float32),
        mesh=mesh,
        scratch_types=[
            pltpu.VMEM_SHARED((np_rows, H), jnp.float32),
            pltpu.VMEM((SUPER, CH), jnp.int32),
            pltpu.VMEM((SUPER, CH), jnp.int32),
            pltpu.VMEM((CH, H), jnp.float32),
            pltpu.VMEM((CH, H), jnp.float32),
            pltpu.VMEM((ZR, H), jnp.float32),
            pltpu.SemaphoreType.DMA,
            pltpu.SemaphoreType.DMA,
        ],
    )
    def k(m_hbm, src_hbm, dst_hbm, out_hbm, acc, src_v, dst_v, rows_a, rows_b,
          zbuf, gsem, ssem):
        cid = lax.axis_index("c")
        sid = lax.axis_index("s")
        r0 = pl.multiple_of(sid * rpt, 8)
        nsuper = nchunk // SUPER

        def zrow(i, c):
            def zcol(kk, c2):
                zbuf[i, pl.ds(kk * 16, 16)] = jnp.zeros((16,), jnp.float32)
                return c2
            return lax.fori_loop(0, H // 16, zcol, c)
        lax.fori_loop(0, ZR, zrow, 0)

        for kb in range(bpc):
            b = cid * bpc + kb
            bias = b * np_rows
            zds = []
            off = 0
            while off < rpt:
                sz = min(ZR, rpt - off)
                zds.append(pltpu.async_copy(
                    zbuf.at[pl.ds(0, sz)],
                    acc.at[pl.ds(pl.multiple_of(r0 + off, 8), sz)], ssem))
                off += sz
            for d in zds:
                d.wait()
            plsc.subcore_barrier()

            def super_body(s, carry):
                s0 = pl.multiple_of(s * SUPER, 8)
                pltpu.sync_copy(src_hbm.at[sid, pl.ds(s0, SUPER)], src_v)
                pltpu.sync_copy(dst_hbm.at[sid, pl.ds(s0, SUPER)], dst_v)

                def bias_j(j, c2):
                    def bias_k(kk, c3):
                        sl = pl.ds(kk * 16, 16)
                        src_v[j, sl] = src_v[j, sl] + bias
                        return c3
                    return lax.fori_loop(0, CH // 16, bias_k, c2)
                lax.fori_loop(0, SUPER, bias_j, carry)

                rows = (rows_a, rows_b)
                g = pltpu.async_copy(m_hbm.at[src_v.at[0]], rows[0], gsem)
                sprev = None
                for jj in range(SUPER):
                    rb = jj % 2
                    g.wait()
                    if sprev is not None:
                        sprev.wait()
                    if jj + 1 < SUPER:
                        g = pltpu.async_copy(m_hbm.at[src_v.at[jj + 1]],
                                             rows[1 - rb], gsem)
                    sprev = pltpu.async_copy(rows[rb], acc.at[dst_v.at[jj]],
                                             ssem, add=True)
                sprev.wait()
                return carry
            lax.fori_loop(0, nsuper, super_body, 0)

            plsc.subcore_barrier()
            rows = (rows_a, rows_b)
            offs = list(range(0, rpt, CH))
            descs = [None, None]
            for t, off in enumerate(offs):
                sz = min(CH, rpt - off)
                rb = t % 2
                if descs[rb] is not None:
                    descs[rb][1].wait()
                pltpu.sync_copy(acc.at[pl.ds(pl.multiple_of(r0 + off, 8), sz)],
                                rows[rb].at[pl.ds(0, sz)])
                d = pltpu.async_copy(
                    rows[rb].at[pl.ds(0, sz)],
                    out_hbm.at[pl.ds(
                        pl.multiple_of(b * np_rows + r0 + off, 8), sz)],
                    gsem)
                descs[rb] = (off, d)
            for e in descs:
                if e is not None:
                    e[1].wait()

    return k(m, src3, dst3)


_F32 = jnp.float32
_BF16 = jnp.bfloat16


def _bdot(a, b):
    return jnp.dot(a.astype(_BF16), b.astype(_BF16),
                   preferred_element_type=_F32)


def _tc_prologue(nodes_pp, cov_flat, cw0, p_rows, n_batch, bm_rows):
    H = nodes_pp.shape[1]
    nb = p_rows // bm_rows
    grid = (n_batch * nb,)
    R = n_batch * p_rows

    def body(nodes_ref, cov_ref, cw_ref, x0_ref, m0_ref):
        lane = lax.broadcasted_iota(jnp.int32, (bm_rows, H), 1)
        xb = jnp.where(lane == H - 1, cov_ref[...], nodes_ref[...])
        x0_ref[...] = xb
        m0_ref[...] = _bdot(xb, cw_ref[...])

    return pl.pallas_call(
        body,
        grid=grid,
        in_specs=[
            pl.BlockSpec((bm_rows, H), lambda i: (i % nb, 0)),
            pl.BlockSpec((bm_rows, 1), lambda i: (i, 0)),
            pl.BlockSpec((H, H), lambda i: (0, 0)),
        ],
        out_specs=[
            pl.BlockSpec((bm_rows, H), lambda i: (i, 0)),
            pl.BlockSpec((bm_rows, H), lambda i: (i, 0)),
        ],
        out_shape=[
            jax.ShapeDtypeStruct((R, H), _F32),
            jax.ShapeDtypeStruct((R, H), _F32),
        ],
    )(nodes_pp, cov_flat, cw0)


def _gru(ma, xb, wi, wh, bi, bh, H):
    gi = _bdot(ma, wi) + bi
    gh = _bdot(xb, wh) + bh
    r = jax.nn.sigmoid(gi[:, :H] + gh[:, :H])
    z = jax.nn.sigmoid(gi[:, H:2 * H] + gh[:, H:2 * H])
    n = jnp.tanh(gi[:, 2 * H:] + r * gh[:, 2 * H:])
    return (1.0 - z) * n + z * xb


def _tc_gru_conv(m_agg, x, wi, wh, bi, bh, cw_next, bm_rows):
    BN, H = x.shape
    grid = (BN // bm_rows,)

    def body(ma_ref, x_ref, wi_ref, wh_ref, bi_ref, bh_ref, cw_ref,
             xo_ref, mo_ref):
        xn = _gru(ma_ref[...], x_ref[...], wi_ref[...], wh_ref[...],
                  bi_ref[...], bh_ref[...], H)
        xo_ref[...] = xn
        mo_ref[...] = _bdot(xn, cw_ref[...])

    row_spec = pl.BlockSpec((bm_rows, H), lambda i: (i, 0))
    full = lambda shp: pl.BlockSpec(shp, lambda i: (0, 0))
    return pl.pallas_call(
        body,
        grid=grid,
        in_specs=[
            row_spec, row_spec,
            full((H, 3 * H)), full((H, 3 * H)),
            full((1, 3 * H)), full((1, 3 * H)),
            full((H, H)),
        ],
        out_specs=[row_spec, row_spec],
        out_shape=[
            jax.ShapeDtypeStruct((BN, H), _F32),
            jax.ShapeDtypeStruct((BN, H), _F32),
        ],
    )(m_agg, x, wi, wh, bi, bh, cw_next)


def _tc_final(m_agg, x, x0, wi, wh, bi, bh, Wm, bm, Wi, bi2, Wj, bj2,
              Wv, bv, Wa, ba, n_nodes, p_rows, n_batch, bm_rows):
    BN, H = x.shape
    A = Wa.shape[1]
    nb = p_rows // bm_rows
    grid = (BN // bm_rows,)
    nsteps = BN // bm_rows

    def body(ma_ref, x_ref, x0_ref, wi_ref, wh_ref, bi_ref, bh_ref,
             wm_ref, bm_ref, wi2_ref, bi2_ref, wj_ref, bj2_ref,
             wv_ref, bv_ref, wa_ref, ba_ref, out_ref, pooled_ref):
        i = pl.program_id(0)

        @pl.when(i == 0)
        def _():
            pooled_ref[...] = jnp.zeros_like(pooled_ref)

        xn = _gru(ma_ref[...], x_ref[...], wi_ref[...], wh_ref[...],
                  bi_ref[...], bh_ref[...], H)
        h = jnp.maximum(_bdot(xn, wm_ref[...]) + bm_ref[...], 0.0)
        x0b = x0_ref[...]
        a1 = (_bdot(h, wi2_ref[:H, :]) + _bdot(x0b, wi2_ref[H:, :])
              + bi2_ref[...])
        a2 = (_bdot(h, wj_ref[:H, :]) + _bdot(x0b, wj_ref[H:, :])
              + bj2_ref[...])
        att = jax.nn.sigmoid(a1) * jnp.maximum(a2, 0.0)
        kb = i % nb
        nloc = kb * bm_rows + lax.broadcasted_iota(jnp.int32, att.shape, 0)
        att = jnp.where(nloc < n_nodes, att, 0.0)
        bsum = jnp.sum(att, axis=0, keepdims=True)
        bidx = i // nb
        pooled_ref[pl.ds(bidx, 1), :] = pooled_ref[pl.ds(bidx, 1), :] + bsum

        @pl.when(i == nsteps - 1)
        def _():
            pooled = jnp.maximum(pooled_ref[...], 0.0)
            value = jnp.dot(pooled, wv_ref[...],
                            preferred_element_type=_F32) + bv_ref[...]
            adv = jnp.dot(pooled, wa_ref[...],
                          preferred_element_type=_F32) + ba_ref[...]
            row = lax.broadcasted_iota(jnp.int32, adv.shape, 0)
            adv_mean = jnp.sum(jnp.where(row < n_batch, adv, 0.0)) / (
                n_batch * A)
            out_ref[...] = value + adv - adv_mean

    row_spec = pl.BlockSpec((bm_rows, H), lambda i: (i, 0))
    full = lambda shp: pl.BlockSpec(shp, lambda i: (0, 0))
    out8 = pl.pallas_call(
        body,
        grid=grid,
        in_specs=[
            row_spec, row_spec, row_spec,
            full((H, 3 * H)), full((H, 3 * H)),
            full((1, 3 * H)), full((1, 3 * H)),
            full((H, H)), full((1, H)),
            full((2 * H, 2 * H)), full((1, 2 * H)),
            full((2 * H, 2 * H)), full((1, 2 * H)),
            full((2 * H, 1)), full((1, 1)),
            full((2 * H, A)), full((1, A)),
        ],
        out_specs=full((8, A)),
        out_shape=jax.ShapeDtypeStruct((8, A), _F32),
        scratch_shapes=[pltpu.VMEM((8, 2 * H), _F32)],
    )(m_agg, x, x0, wi, wh, bi, bh, Wm, bm, Wi, bi2, Wj, bj2, Wv, bv, Wa, ba)
    return out8


def kernel(input, nodes, edges, count2label, conv_w, gru_wi, gru_wh,
           gru_bi, gru_bh, Wi, bi, Wj, bj, Wm, bm, Wv, bv, Wa, ba):
    Bn = input.shape[0]
    Cn = count2label.shape[0]
    Nn, F = nodes.shape
    H = F + 1
    E = edges.shape[1]
    Lc = conv_w.shape[0]

    np_rows = NT * (-(-(-(-Nn // NT)) // 8) * 8)
    BM = np_rows // NT

    cov = input[:, :Cn]
    winner = jnp.zeros((Nn,), jnp.int32).at[count2label].max(
        jnp.arange(1, Cn + 1, dtype=jnp.int32))
    covAll = jnp.where(winner[None, :] > 0,
                       jnp.take(cov, jnp.maximum(winner - 1, 0), axis=1),
                       0.0)
    cov_flat = jnp.pad(covAll, ((0, 0), (0, np_rows - Nn))).reshape(
        Bn * np_rows, 1)
    nodes_pp = jnp.pad(nodes, ((0, np_rows - Nn), (0, 1)))

    ept = E // NT
    nchunk = -(-(-(-ept // CH)) // SUPER) * SUPER
    npad = nchunk * CH - ept
    src = edges[0].reshape(NT, ept)
    dst = edges[1].reshape(NT, ept)
    if npad:
        pad_src = ((jnp.arange(NT, dtype=jnp.int32)[:, None] * 1259
                    + jnp.arange(npad, dtype=jnp.int32)[None, :] * 631) % Nn)
        pad_dst = (Nn + jnp.arange(NT, dtype=jnp.int32)[:, None]
                   + jnp.zeros((1, npad), jnp.int32))
        src = jnp.concatenate([src, pad_src], axis=1)
        dst = jnp.concatenate([dst, pad_dst], axis=1)
    src3 = src.reshape(NT, nchunk, CH)
    dst3 = dst.reshape(NT, nchunk, CH)

    bi1 = gru_bi.reshape(1, 3 * H)
    bh1 = gru_bh.reshape(1, 3 * H)

    x0, m = _tc_prologue(nodes_pp, cov_flat, conv_w[0], np_rows, Bn, BM)
    x = x0
    for i in range(Lc):
        m_agg = _seg_sum_sc(m, src3, dst3, np_rows, Bn)
        if i < Lc - 1:
            x, m = _tc_gru_conv(m_agg, x, gru_wi, gru_wh, bi1, bh1,
                                conv_w[i + 1], BM)
        else:
            out8 = _tc_final(m_agg, x, x0, gru_wi, gru_wh, bi1, bh1,
                             Wm, bm.reshape(1, H),
                             Wi, bi.reshape(1, 2 * H),
                             Wj, bj.reshape(1, 2 * H),
                             Wv, bv.reshape(1, 1),
                             Wa, ba.reshape(1, -1),
                             Nn, np_rows, Bn, BM)
    return out8[:Bn]


# speedup vs baseline: 12.2375x; 1.0041x over previous
import functools

import jax
import jax.numpy as jnp
from jax import lax
from jax.experimental import pallas as pl
from jax.experimental.pallas import tpu as pltpu
from jax.experimental.pallas import tpu_sc as plsc

NT = 16
NCC = 2
CH = 125
SUPER = 32
ZR = 64
DR = 120


def _seg_sum_sc(m, src4, dst3, np_rows, n_batch):
    BN, H = m.shape
    nchunk = dst3.shape[1]
    rpt = np_rows // NT
    bpc = n_batch // NCC

    mesh = plsc.VectorSubcoreMesh(core_axis_name="c", subcore_axis_name="s",
                                  num_cores=NCC, num_subcores=NT)

    @functools.partial(
        pl.kernel,
        out_type=jax.ShapeDtypeStruct((BN, H), jnp.float32),
        mesh=mesh,
        scratch_types=[
            pltpu.VMEM_SHARED((np_rows, H), jnp.float32),
            pltpu.VMEM((SUPER, CH), jnp.int32),
            pltpu.VMEM((SUPER, CH), jnp.int32),
            pltpu.VMEM((CH, H), jnp.float32),
            pltpu.VMEM((CH, H), jnp.float32),
            pltpu.VMEM((ZR, H), jnp.float32),
            pltpu.SemaphoreType.DMA,
            pltpu.SemaphoreType.DMA,
        ],
    )
    def k(m_hbm, src_hbm, dst_hbm, out_hbm, acc, src_v, dst_v, rows_a, rows_b,
          zbuf, gsem, ssem):
        cid = lax.axis_index("c")
        sid = lax.axis_index("s")
        r0 = pl.multiple_of(sid * rpt, 8)
        nsuper = nchunk // SUPER

        def zrow(i, c):
            def zcol(kk, c2):
                zbuf[i, pl.ds(kk * 16, 16)] = jnp.zeros((16,), jnp.float32)
                return c2
            return lax.fori_loop(0, H // 16, zcol, c)
        lax.fori_loop(0, ZR, zrow, 0)

        zds = []
        off = 0
        while off < rpt:
            sz = min(ZR, rpt - off)
            zds.append(pltpu.async_copy(
                zbuf.at[pl.ds(0, sz)],
                acc.at[pl.ds(pl.multiple_of(r0 + off, 8), sz)], ssem))
            off += sz
        for d in zds:
            d.wait()

        for kb in range(bpc):
            b = cid * bpc + kb
            plsc.subcore_barrier()

            def super_body(s, carry):
                s0 = pl.multiple_of(s * SUPER, 8)
                pltpu.sync_copy(src_hbm.at[b, sid, pl.ds(s0, SUPER)], src_v)
                pltpu.sync_copy(dst_hbm.at[sid, pl.ds(s0, SUPER)], dst_v)

                rows = (rows_a, rows_b)
                g = pltpu.async_copy(m_hbm.at[src_v.at[0]], rows[0], gsem)
                sprev = None
                for jj in range(SUPER):
                    rb = jj % 2
                    g.wait()
                    if sprev is not None:
                        sprev.wait()
                    if jj + 1 < SUPER:
                        g = pltpu.async_copy(m_hbm.at[src_v.at[jj + 1]],
                                             rows[1 - rb], gsem)
                    sprev = pltpu.async_copy(rows[rb], acc.at[dst_v.at[jj]],
                                             ssem, add=True)
                sprev.wait()
                return carry
            lax.fori_loop(0, nsuper, super_body, 0)

            plsc.subcore_barrier()
            rows = (rows_a, rows_b)
            descs = [None, None]
            zds = []
            for t, off in enumerate(range(0, rpt, DR)):
                sz = min(DR, rpt - off)
                rb = t % 2
                if descs[rb] is not None:
                    descs[rb].wait()
                a0 = pl.multiple_of(r0 + off, 8)
                pltpu.sync_copy(acc.at[pl.ds(a0, sz)],
                                rows[rb].at[pl.ds(0, sz)])
                descs[rb] = pltpu.async_copy(
                    rows[rb].at[pl.ds(0, sz)],
                    out_hbm.at[pl.ds(
                        pl.multiple_of(b * np_rows + r0 + off, 8), sz)],
                    gsem)
                zoff = 0
                while zoff < sz:
                    zsz = min(ZR, sz - zoff)
                    zds.append(pltpu.async_copy(
                        zbuf.at[pl.ds(0, zsz)],
                        acc.at[pl.ds(pl.multiple_of(a0 + zoff, 8), zsz)],
                        ssem))
                    zoff += zsz
            for e in descs:
                if e is not None:
                    e.wait()
            for d in zds:
                d.wait()

    return k(m, src4, dst3)


_F32 = jnp.float32
_BF16 = jnp.bfloat16


def _bdot(a, b):
    return jnp.dot(a.astype(_BF16), b.astype(_BF16),
                   preferred_element_type=_F32)


def _tc_prologue(nodes_pp, cov_flat, cw0, p_rows, n_batch, bm_rows):
    H = nodes_pp.shape[1]
    nb = p_rows // bm_rows
    grid = (n_batch * nb,)
    R = n_batch * p_rows

    def body(nodes_ref, cov_ref, cw_ref, x0_ref, m0_ref):
        lane = lax.broadcasted_iota(jnp.int32, (bm_rows, H), 1)
        xb = jnp.where(lane == H - 1, cov_ref[...], nodes_ref[...])
        x0_ref[...] = xb
        m0_ref[...] = _bdot(xb, cw_ref[...])

    return pl.pallas_call(
        body,
        grid=grid,
        in_specs=[
            pl.BlockSpec((bm_rows, H), lambda i: (i % nb, 0)),
            pl.BlockSpec((bm_rows, 1), lambda i: (i, 0)),
            pl.BlockSpec((H, H), lambda i: (0, 0)),
        ],
        out_specs=[
            pl.BlockSpec((bm_rows, H), lambda i: (i, 0)),
            pl.BlockSpec((bm_rows, H), lambda i: (i, 0)),
        ],
        out_shape=[
            jax.ShapeDtypeStruct((R, H), _F32),
            jax.ShapeDtypeStruct((R, H), _F32),
        ],
    )(nodes_pp, cov_flat, cw0)


def _gru(ma, xb, wi, wh, bi, bh, H):
    gi = _bdot(ma, wi) + bi
    gh = _bdot(xb, wh) + bh
    r = jax.nn.sigmoid(gi[:, :H] + gh[:, :H])
    z = jax.nn.sigmoid(gi[:, H:2 * H] + gh[:, H:2 * H])
    n = jnp.tanh(gi[:, 2 * H:] + r * gh[:, 2 * H:])
    return (1.0 - z) * n + z * xb


def _tc_gru_conv(m_agg, x, wi, wh, bi, bh, cw_next, bm_rows):
    BN, H = x.shape
    grid = (BN // bm_rows,)

    def body(ma_ref, x_ref, wi_ref, wh_ref, bi_ref, bh_ref, cw_ref,
             xo_ref, mo_ref):
        xn = _gru(ma_ref[...], x_ref[...], wi_ref[...], wh_ref[...],
                  bi_ref[...], bh_ref[...], H)
        xo_ref[...] = xn
        mo_ref[...] = _bdot(xn, cw_ref[...])

    row_spec = pl.BlockSpec((bm_rows, H), lambda i: (i, 0))
    full = lambda shp: pl.BlockSpec(shp, lambda i: (0, 0))
    return pl.pallas_call(
        body,
        grid=grid,
        in_specs=[
            row_spec, row_spec,
            full((H, 3 * H)), full((H, 3 * H)),
            full((1, 3 * H)), full((1, 3 * H)),
            full((H, H)),
        ],
        out_specs=[row_spec, row_spec],
        out_shape=[
            jax.ShapeDtypeStruct((BN, H), _F32),
            jax.ShapeDtypeStruct((BN, H), _F32),
        ],
    )(m_agg, x, wi, wh, bi, bh, cw_next)


def _tc_final(m_agg, x, x0, wi, wh, bi, bh, Wm, bm, Wi, bi2, Wj, bj2,
              Wv, bv, Wa, ba, n_nodes, p_rows, n_batch, bm_rows):
    BN, H = x.shape
    A = Wa.shape[1]
    nb = p_rows // bm_rows
    grid = (BN // bm_rows,)
    nsteps = BN // bm_rows

    def body(ma_ref, x_ref, x0_ref, wi_ref, wh_ref, bi_ref, bh_ref,
             wm_ref, bm_ref, wi2_ref, bi2_ref, wj_ref, bj2_ref,
             wv_ref, bv_ref, wa_ref, ba_ref, out_ref, pooled_ref):
        i = pl.program_id(0)

        @pl.when(i == 0)
        def _():
            pooled_ref[...] = jnp.zeros_like(pooled_ref)

        xn = _gru(ma_ref[...], x_ref[...], wi_ref[...], wh_ref[...],
                  bi_ref[...], bh_ref[...], H)
        h = jnp.maximum(_bdot(xn, wm_ref[...]) + bm_ref[...], 0.0)
        x0b = x0_ref[...]
        a1 = (_bdot(h, wi2_ref[:H, :]) + _bdot(x0b, wi2_ref[H:, :])
              + bi2_ref[...])
        a2 = (_bdot(h, wj_ref[:H, :]) + _bdot(x0b, wj_ref[H:, :])
              + bj2_ref[...])
        att = jax.nn.sigmoid(a1) * jnp.maximum(a2, 0.0)
        kb = i % nb
        nloc = kb * bm_rows + lax.broadcasted_iota(jnp.int32, att.shape, 0)
        att = jnp.where(nloc < n_nodes, att, 0.0)
        bsum = jnp.sum(att, axis=0, keepdims=True)
        bidx = i // nb
        pooled_ref[pl.ds(bidx, 1), :] = pooled_ref[pl.ds(bidx, 1), :] + bsum

        @pl.when(i == nsteps - 1)
        def _():
            pooled = jnp.maximum(pooled_ref[...], 0.0)
            value = jnp.dot(pooled, wv_ref[...],
                            preferred_element_type=_F32) + bv_ref[...]
            adv = jnp.dot(pooled, wa_ref[...],
                          preferred_element_type=_F32) + ba_ref[...]
            row = lax.broadcasted_iota(jnp.int32, adv.shape, 0)
            adv_mean = jnp.sum(jnp.where(row < n_batch, adv, 0.0)) / (
                n_batch * A)
            out_ref[...] = value + adv - adv_mean

    row_spec = pl.BlockSpec((bm_rows, H), lambda i: (i, 0))
    full = lambda shp: pl.BlockSpec(shp, lambda i: (0, 0))
    out8 = pl.pallas_call(
        body,
        grid=grid,
        in_specs=[
            row_spec, row_spec, row_spec,
            full((H, 3 * H)), full((H, 3 * H)),
            full((1, 3 * H)), full((1, 3 * H)),
            full((H, H)), full((1, H)),
            full((2 * H, 2 * H)), full((1, 2 * H)),
            full((2 * H, 2 * H)), full((1, 2 * H)),
            full((2 * H, 1)), full((1, 1)),
            full((2 * H, A)), full((1, A)),
        ],
        out_specs=full((8, A)),
        out_shape=jax.ShapeDtypeStruct((8, A), _F32),
        scratch_shapes=[pltpu.VMEM((8, 2 * H), _F32)],
    )(m_agg, x, x0, wi, wh, bi, bh, Wm, bm, Wi, bi2, Wj, bj2, Wv, bv, Wa, ba)
    return out8


def kernel(input, nodes, edges, count2label, conv_w, gru_wi, gru_wh,
           gru_bi, gru_bh, Wi, bi, Wj, bj, Wm, bm, Wv, bv, Wa, ba):
    Bn = input.shape[0]
    Cn = count2label.shape[0]
    Nn, F = nodes.shape
    H = F + 1
    E = edges.shape[1]
    Lc = conv_w.shape[0]

    np_rows = NT * (-(-(-(-Nn // NT)) // 8) * 8)
    BM = np_rows // NT

    cov = input[:, :Cn]
    winner = jnp.zeros((Nn,), jnp.int32).at[count2label].max(
        jnp.arange(1, Cn + 1, dtype=jnp.int32))
    covAll = jnp.where(winner[None, :] > 0,
                       jnp.take(cov, jnp.maximum(winner - 1, 0), axis=1),
                       0.0)
    cov_flat = jnp.pad(covAll, ((0, 0), (0, np_rows - Nn))).reshape(
        Bn * np_rows, 1)
    nodes_pp = jnp.pad(nodes, ((0, np_rows - Nn), (0, 1)))

    ept = E // NT
    nchunk = -(-(-(-ept // CH)) // SUPER) * SUPER
    npad = nchunk * CH - ept
    src = edges[0].reshape(NT, ept)
    dst = edges[1].reshape(NT, ept)
    if npad:
        pad_src = ((jnp.arange(NT, dtype=jnp.int32)[:, None] * 1259
                    + jnp.arange(npad, dtype=jnp.int32)[None, :] * 631) % Nn)
        pad_dst = (Nn + jnp.arange(NT, dtype=jnp.int32)[:, None]
                   + jnp.zeros((1, npad), jnp.int32))
        src = jnp.concatenate([src, pad_src], axis=1)
        dst = jnp.concatenate([dst, pad_dst], axis=1)
    src3 = src.reshape(NT, nchunk, CH)
    dst3 = dst.reshape(NT, nchunk, CH)
    src4 = src3[None] + (jnp.arange(Bn, dtype=jnp.int32)
                         * np_rows)[:, None, None, None]

    bi1 = gru_bi.reshape(1, 3 * H)
    bh1 = gru_bh.reshape(1, 3 * H)

    x0, m = _tc_prologue(nodes_pp, cov_flat, conv_w[0], np_rows, Bn, BM)
    x = x0
    for i in range(Lc):
        m_agg = _seg_sum_sc(m, src4, dst3, np_rows, Bn)
        if i < Lc - 1:
            x, m = _tc_gru_conv(m_agg, x, gru_wi, gru_wh, bi1, bh1,
                                conv_w[i + 1], BM)
        else:
            out8 = _tc_final(m_agg, x, x0, gru_wi, gru_wh, bi1, bh1,
                             Wm, bm.reshape(1, H),
                             Wi, bi.reshape(1, 2 * H),
                             Wj, bj.reshape(1, 2 * H),
                             Wv, bv.reshape(1, 1),
                             Wa, ba.reshape(1, -1),
                             Nn, np_rows, Bn, BM)
    return out8[:Bn]
